# baseline (speedup 1.0000x reference)
import functools

import jax
import jax.numpy as jnp
from jax import lax
from jax.experimental import pallas as pl
from jax.experimental.pallas import tpu as pltpu
from jax.experimental.pallas import tpu_sc as plsc

EMBED = 64
BATCH = 16384
NC = 2
NS = 16
NW = NC * NS
BPW = BATCH // NW
CH = 128
NCH = BPW // CH
NV = EMBED // 16


def _sc_body(idx_u, idx_v, idx_n, center2, context2,
             neg_out, pos_out,
             iu_s, iv_s, in_s,
             ru, rv, rn, negacc, posbuf,
             semu, semv, semn):
    wid = lax.axis_index("s") * NC + lax.axis_index("c")

    pltpu.sync_copy(idx_u.at[wid], iu_s)
    pltpu.sync_copy(idx_v.at[wid], iv_s)
    pltpu.sync_copy(idx_n.at[wid], in_s)

    zero16 = jnp.zeros((16,), jnp.float32)
    for j in range(8):
        posbuf[0, pl.ds(16 * j, 16)] = zero16

    def chunk(c, posv):
        cbase = c * CH

        def fire(g, _):
            uvec = iu_s[pl.ds(cbase + g * 16, 16)]
            vvec = iv_s[pl.ds(cbase + g * 16, 16)]
            nvec = in_s[pl.ds(cbase + g * 16, 16)]
            for l in range(16):
                dst = pl.ds(g * 16 + l, 1)
                pltpu.make_async_copy(
                    center2.at[pl.ds(uvec[l] >> 1, 1)], ru.at[dst], semu).start()
                pltpu.make_async_copy(
                    context2.at[pl.ds(vvec[l] >> 1, 1)], rv.at[dst], semv).start()
                pltpu.make_async_copy(
                    context2.at[pl.ds(nvec[l] >> 1, 1)], rn.at[dst], semn).start()
            return 0

        lax.fori_loop(0, CH // 16, fire, 0)
        pltpu.make_async_copy(center2.at[pl.ds(0, CH)], ru, semu).wait()
        pltpu.make_async_copy(context2.at[pl.ds(0, CH)], rv, semv).wait()
        pltpu.make_async_copy(context2.at[pl.ds(0, CH)], rn, semn).wait()

        def rowgrp(g, p):
            uvec = iu_s[pl.ds(cbase + g * 16, 16)]
            vvec = iv_s[pl.ds(cbase + g * 16, 16)]
            nvec = in_s[pl.ds(cbase + g * 16, 16)]
            for k in range(16):
                i = g * 16 + k
                cu = (uvec[k] & 1) == 1
                cv = (vvec[k] & 1) == 1
                cn = (nvec[k] & 1) == 1
                s = None
                for j in range(NV):
                    dlo = pl.ds(16 * j, 16)
                    dhi = pl.ds(64 + 16 * j, 16)
                    u = jnp.where(cu, ru[i, dhi], ru[i, dlo])
                    v = jnp.where(cv, rv[i, dhi], rv[i, dlo])
                    n = jnp.where(cn, rn[i, dhi], rn[i, dlo])
                    p = p + u * v
                    s = u * n if s is None else s + u * n
                negacc[cbase // 8 + g * 2 + k // 8, pl.ds((k % 8) * 16, 16)] = s
            return p

        return lax.fori_loop(0, CH // 16, rowgrp, posv)

    posv = lax.fori_loop(0, NCH, chunk, zero16)
    posbuf[0, pl.ds(0, 16)] = posv
    pltpu.sync_copy(negacc, neg_out.at[pl.ds(wid * (BPW // 8), BPW // 8)])
    pltpu.sync_copy(posbuf, pos_out.at[pl.ds(wid, 1)])


def _tc_body(neg_ref, pos_ref, out_ref):
    pos_total = jnp.sum(pos_ref[...])
    pos_loss = jax.nn.log_sigmoid(pos_total)
    lane = lax.broadcasted_iota(jnp.int32, (128, 8), 0)
    grp = lax.broadcasted_iota(jnp.int32, (128, 8), 1)
    sel = jnp.where(lane // 16 == grp, 1.0, 0.0).astype(jnp.float32)
    neg_score = jnp.dot(neg_ref[...], sel,
                        preferred_element_type=jnp.float32)
    neg_loss = jnp.sum(jax.nn.log_sigmoid(-neg_score))
    out_ref[0, 0] = -(pos_loss + neg_loss)


def _sc_call(pos_u, pos_v, neg_v, center_table, context_table):
    iu = pos_u.reshape(NW, BPW)
    iv = pos_v.reshape(NW, BPW)
    inn = neg_v.reshape(NW, BPW)
    c2 = center_table.reshape(-1, 128)
    x2 = context_table.reshape(-1, 128)

    sc = functools.partial(
        pl.kernel,
        mesh=plsc.VectorSubcoreMesh(core_axis_name="c", subcore_axis_name="s"),
        compiler_params=pltpu.CompilerParams(use_tc_tiling_on_sc=True),
        out_type=[
            jax.ShapeDtypeStruct((BATCH // 8, 128), jnp.float32),
            jax.ShapeDtypeStruct((NW, 128), jnp.float32),
        ],
        scratch_types=[
            pltpu.VMEM((BPW,), jnp.int32),
            pltpu.VMEM((BPW,), jnp.int32),
            pltpu.VMEM((BPW,), jnp.int32),
            pltpu.VMEM((CH, 128), jnp.float32),
            pltpu.VMEM((CH, 128), jnp.float32),
            pltpu.VMEM((CH, 128), jnp.float32),
            pltpu.VMEM((BPW // 8, 128), jnp.float32),
            pltpu.VMEM((1, 128), jnp.float32),
            pltpu.SemaphoreType.DMA,
            pltpu.SemaphoreType.DMA,
            pltpu.SemaphoreType.DMA,
        ],
    )(_sc_body)
    return sc(iu, iv, inn, c2, x2)


@jax.jit
def kernel(pos_u, pos_v, neg_v, center_table, context_table):
    neg2, posp = _sc_call(pos_u, pos_v, neg_v, center_table, context_table)

    out = pl.pallas_call(
        _tc_body,
        out_shape=jax.ShapeDtypeStruct((1, 1), jnp.float32),
        out_specs=pl.BlockSpec(memory_space=pltpu.SMEM),
    )(neg2, posp)
    return out[0, 0]


# speedup vs baseline: 1.5679x; 1.5679x over previous
import functools

import jax
import jax.numpy as jnp
from jax import lax
from jax.experimental import pallas as pl
from jax.experimental.pallas import tpu as pltpu
from jax.experimental.pallas import tpu_sc as plsc

EMBED = 64
BATCH = 16384
NC = 2
NS = 16
NW = NC * NS
BPW = BATCH // NW
CH = 128
NCH = BPW // CH
NV = EMBED // 16


def _sc_body(idx_u, idx_v, idx_n, center, context,
             neg_out, pos_out,
             iu_s, iv_s, in_s,
             ru, rv, rn, negacc, posbuf,
             semu, semv, semn):
    wid = lax.axis_index("s") * NC + lax.axis_index("c")

    pltpu.sync_copy(idx_u.at[wid], iu_s)
    pltpu.sync_copy(idx_v.at[wid], iv_s)
    pltpu.sync_copy(idx_n.at[wid], in_s)

    zero16 = jnp.zeros((16,), jnp.float32)
    for j in range(8):
        posbuf[0, pl.ds(16 * j, 16)] = zero16

    def chunk(c, posv):
        cbase = c * CH

        def fire(g, _):
            uvec = iu_s[pl.ds(cbase + g * 16, 16)]
            vvec = iv_s[pl.ds(cbase + g * 16, 16)]
            nvec = in_s[pl.ds(cbase + g * 16, 16)]
            for l in range(16):
                dst = pl.ds(g * 16 + l, 1)
                pltpu.make_async_copy(
                    center.at[pl.ds(uvec[l], 1)], ru.at[dst], semu).start()
                pltpu.make_async_copy(
                    context.at[pl.ds(vvec[l], 1)], rv.at[dst], semv).start()
                pltpu.make_async_copy(
                    context.at[pl.ds(nvec[l], 1)], rn.at[dst], semn).start()
            return 0

        lax.fori_loop(0, CH // 16, fire, 0)
        pltpu.make_async_copy(center.at[pl.ds(0, CH)], ru, semu).wait()
        pltpu.make_async_copy(context.at[pl.ds(0, CH)], rv, semv).wait()
        pltpu.make_async_copy(context.at[pl.ds(0, CH)], rn, semn).wait()

        def rowblk(t, p):
            for k in range(8):
                i = t * 8 + k
                s = None
                for j in range(NV):
                    d = pl.ds(16 * j, 16)
                    u = ru[i, d]
                    v = rv[i, d]
                    n = rn[i, d]
                    p = p + u * v
                    s = u * n if s is None else s + u * n
                negacc[cbase // 8 + t, pl.ds(k * 16, 16)] = s
            return p

        return lax.fori_loop(0, CH // 8, rowblk, posv)

    posv = lax.fori_loop(0, NCH, chunk, zero16)
    posbuf[0, pl.ds(0, 16)] = posv
    pltpu.sync_copy(negacc, neg_out.at[pl.ds(wid * (BPW // 8), BPW // 8)])
    pltpu.sync_copy(posbuf, pos_out.at[pl.ds(wid, 1)])


def _tc_body(neg_ref, pos_ref, out_ref):
    pos_total = jnp.sum(pos_ref[...])
    pos_loss = jax.nn.log_sigmoid(pos_total)
    lane = lax.broadcasted_iota(jnp.int32, (128, 8), 0)
    grp = lax.broadcasted_iota(jnp.int32, (128, 8), 1)
    sel = jnp.where(lane // 16 == grp, 1.0, 0.0).astype(jnp.float32)
    neg_score = jnp.dot(neg_ref[...], sel,
                        preferred_element_type=jnp.float32)
    neg_loss = jnp.sum(jax.nn.log_sigmoid(-neg_score))
    out_ref[0, 0] = -(pos_loss + neg_loss)


def _sc_call(pos_u, pos_v, neg_v, center_table, context_table):
    iu = pos_u.reshape(NW, BPW)
    iv = pos_v.reshape(NW, BPW)
    inn = neg_v.reshape(NW, BPW)

    sc = functools.partial(
        pl.kernel,
        mesh=plsc.VectorSubcoreMesh(core_axis_name="c", subcore_axis_name="s"),
        compiler_params=pltpu.CompilerParams(use_tc_tiling_on_sc=True),
        out_type=[
            jax.ShapeDtypeStruct((BATCH // 8, 128), jnp.float32),
            jax.ShapeDtypeStruct((NW, 128), jnp.float32),
        ],
        scratch_types=[
            pltpu.VMEM((BPW,), jnp.int32),
            pltpu.VMEM((BPW,), jnp.int32),
            pltpu.VMEM((BPW,), jnp.int32),
            pltpu.VMEM((CH, EMBED), jnp.float32),
            pltpu.VMEM((CH, EMBED), jnp.float32),
            pltpu.VMEM((CH, EMBED), jnp.float32),
            pltpu.VMEM((BPW // 8, 128), jnp.float32),
            pltpu.VMEM((1, 128), jnp.float32),
            pltpu.SemaphoreType.DMA,
            pltpu.SemaphoreType.DMA,
            pltpu.SemaphoreType.DMA,
        ],
    )(_sc_body)
    return sc(iu, iv, inn, center_table, context_table)


@jax.jit
def kernel(pos_u, pos_v, neg_v, center_table, context_table):
    neg2, posp = _sc_call(pos_u, pos_v, neg_v, center_table, context_table)

    out = pl.pallas_call(
        _tc_body,
        out_shape=jax.ShapeDtypeStruct((1, 1), jnp.float32),
        out_specs=pl.BlockSpec(memory_space=pltpu.SMEM),
    )(neg2, posp)
    return out[0, 0]
